# baseline (speedup 1.0000x reference)
import functools

import jax
import jax.numpy as jnp
from jax import lax
from jax.experimental import pallas as pl
from jax.experimental.pallas import tpu as pltpu
from jax.experimental.pallas import tpu_sc as plsc

_NC = 2
_NS = 16
_NW = _NC * _NS
_C = 2


def kernel(x, emb):
    (B,) = x.shape
    V, D = emb.shape
    bpw = B // _NW
    nchunk = bpw // _C

    x2 = x.reshape(_NW, nchunk, _C).astype(jnp.int32)

    mesh = plsc.VectorSubcoreMesh(core_axis_name="c", subcore_axis_name="s")

    @functools.partial(
        pl.kernel,
        out_type=jax.ShapeDtypeStruct((B, D), emb.dtype),
        mesh=mesh,
        scratch_types=[
            pltpu.VMEM((nchunk, _C), jnp.int32),
            pltpu.VMEM((_C, D), emb.dtype),
            pltpu.VMEM((_C, D), emb.dtype),
            pltpu.VMEM((_C, D), emb.dtype),
            pltpu.VMEM((_C, D), emb.dtype),
            pltpu.SemaphoreType.DMA,
            pltpu.SemaphoreType.DMA,
        ],
    )
    def gather_k(x_hbm, emb_hbm, out_hbm, idx_v, b0, b1, b2, b3, sg, sw):
        wid = lax.axis_index("s") * _NC + lax.axis_index("c")
        rbase = wid * bpw
        bufs = (b0, b1, b2, b3)

        pltpu.sync_copy(x_hbm.at[wid], idx_v)
        pltpu.async_copy(emb_hbm.at[idx_v.at[0]], b0, sg)

        @pl.loop(0, nchunk, step=4)
        def _(base):
            for t in range(4):
                k = base + t
                b = bufs[t]
                nxt = bufs[(t + 1) % 4]
                prv = bufs[(t + 3) % 4]

                pltpu.make_async_copy(emb_hbm.at[idx_v.at[k]], b, sg).wait()

                @pl.when(k + 1 < nchunk)
                def _():
                    pltpu.async_copy(emb_hbm.at[idx_v.at[k + 1]], nxt, sg)

                @pl.when(k >= 1)
                def _():
                    pltpu.make_async_copy(
                        prv, out_hbm.at[pl.ds(rbase + (k - 1) * _C, _C)], sw
                    ).wait()

                pltpu.async_copy(b, out_hbm.at[pl.ds(rbase + k * _C, _C)], sw)

        pltpu.make_async_copy(
            b3, out_hbm.at[pl.ds(rbase + (nchunk - 1) * _C, _C)], sw
        ).wait()

    return gather_k(x2, emb)


# speedup vs baseline: 1.2360x; 1.2360x over previous
import functools

import jax
import jax.numpy as jnp
from jax import lax
from jax.experimental import pallas as pl
from jax.experimental.pallas import tpu as pltpu
from jax.experimental.pallas import tpu_sc as plsc

_NC = 2
_NS = 16
_NW = _NC * _NS
_C = 4


def kernel(x, emb):
    (B,) = x.shape
    V, D = emb.shape
    bpw = B // _NW
    nchunk = bpw // _C

    x2 = x.reshape(_NW, nchunk, _C).astype(jnp.int32)

    mesh = plsc.VectorSubcoreMesh(core_axis_name="c", subcore_axis_name="s")

    @functools.partial(
        pl.kernel,
        out_type=jax.ShapeDtypeStruct((B, D), emb.dtype),
        mesh=mesh,
        scratch_types=[
            pltpu.VMEM((nchunk, _C), jnp.int32),
            pltpu.VMEM((_C, D), emb.dtype),
            pltpu.VMEM((_C, D), emb.dtype),
            pltpu.VMEM((_C, D), emb.dtype),
            pltpu.SemaphoreType.DMA,
            pltpu.SemaphoreType.DMA,
            pltpu.SemaphoreType.DMA,
            pltpu.SemaphoreType.DMA,
            pltpu.SemaphoreType.DMA,
            pltpu.SemaphoreType.DMA,
        ],
    )
    def gather_k(x_hbm, emb_hbm, out_hbm, idx_v, b0, b1, b2,
                 sg0, sg1, sg2, sw0, sw1, sw2):
        wid = lax.axis_index("s") * _NC + lax.axis_index("c")
        rbase = wid * bpw
        bufs = (b0, b1, b2)
        sgs = (sg0, sg1, sg2)
        sws = (sw0, sw1, sw2)

        pltpu.sync_copy(x_hbm.at[wid], idx_v)
        pltpu.async_copy(emb_hbm.at[idx_v.at[0]], b0, sg0)

        @pl.loop(0, nchunk - 2, step=3)
        def _(base):
            for t in range(3):
                k = base + t
                tn = (t + 1) % 3

                @pl.when(k >= 2)
                def _():
                    pltpu.make_async_copy(
                        bufs[tn],
                        out_hbm.at[pl.ds(rbase + (k - 2) * _C, _C)],
                        sws[tn],
                    ).wait()

                pltpu.async_copy(emb_hbm.at[idx_v.at[k + 1]], bufs[tn], sgs[tn])
                pltpu.make_async_copy(
                    emb_hbm.at[idx_v.at[k]], bufs[t], sgs[t]
                ).wait()
                pltpu.async_copy(
                    bufs[t], out_hbm.at[pl.ds(rbase + k * _C, _C)], sws[t]
                )

        for t, k in ((0, nchunk - 2), (1, nchunk - 1)):
            tn = (t + 1) % 3
            pltpu.make_async_copy(
                bufs[tn], out_hbm.at[pl.ds(rbase + (k - 2) * _C, _C)], sws[tn]
            ).wait()

            @pl.when(k + 1 < nchunk)
            def _():
                pltpu.async_copy(
                    emb_hbm.at[idx_v.at[k + 1]], bufs[tn], sgs[tn]
                )

            pltpu.make_async_copy(
                emb_hbm.at[idx_v.at[k]], bufs[t], sgs[t]
            ).wait()
            pltpu.async_copy(
                bufs[t], out_hbm.at[pl.ds(rbase + k * _C, _C)], sws[t]
            )
        for t, k in ((0, nchunk - 2), (1, nchunk - 1)):
            pltpu.make_async_copy(
                bufs[t], out_hbm.at[pl.ds(rbase + k * _C, _C)], sws[t]
            ).wait()

    return gather_k(x2, emb)
